# Initial kernel scaffold; baseline (speedup 1.0000x reference)
import jax
import jax.numpy as jnp
from jax.experimental import pallas as pl


def kernel(batch_observations):
    raise NotImplementedError("write your pallas kernel here")


# speedup vs baseline: 5.8672x; 5.8672x over previous
import numpy as np
import jax
import jax.numpy as jnp
from jax.experimental import pallas as pl

L = 32
A = 32
B = 4096
N = A + L
C = N * (N - 1) // 2
OBS = 4 + 2 * L + 2 * (A - 1) + (A - 1)
F = 8
NF = N * F

BB = 128
G = B // BB
GPB = BB // 32
EC = 32 * C
ROWS_PER_STEP = 8


def _consts():
    r, c = np.triu_indices(N, 1)
    rc = np.stack([r, c]).astype(np.int32)

    j = np.arange(EC)
    e = j % C
    blocal = j // C
    base = np.empty((2, ROWS_PER_STEP, EC), np.int32)
    for k in range(2):
        for s in range(ROWS_PER_STEP):
            base[k, s] = rc[k][e] + N * blocal + N * 32 * s

    S = np.zeros((OBS, NF), np.float32)
    CST = np.zeros((NF,), np.float32)
    W = np.zeros((3, NF), np.float32)
    W[0, :] = 1.0
    S[2, 0] = 1; S[3, 1] = 1; S[0, 2] = 1; S[1, 3] = 1
    CST[6] = 2.0
    for l in range(L):
        col = F * (1 + l)
        sx, sy = 4 + 2 * l, 5 + 2 * l
        S[2, col] = 1; S[sx, col] = 1
        S[3, col + 1] = 1; S[sy, col + 1] = 1
        S[sx, col + 2] = 1; S[sy, col + 3] = 1
        S[sx, col + 4] = 1; S[sy, col + 5] = 1
        W[0, col + 4] = 0.0; W[1, col + 4] = 1.0
        W[0, col + 5] = 0.0; W[2, col + 5] = 1.0
    for o in range(A - 1):
        col = F * (1 + L + o)
        sx, sy = 4 + 2 * L + 2 * o, 5 + 2 * L + 2 * o
        S[2, col] = 1; S[sx, col] = 1
        S[3, col + 1] = 1; S[sy, col + 1] = 1
        S[sx, col + 2] = 1; S[sy, col + 3] = 1
        S[sx, col + 4] = 1; S[sy, col + 5] = 1
        W[0, col + 4] = 0.0; W[1, col + 4] = 1.0
        W[0, col + 5] = 0.0; W[2, col + 5] = 1.0
        CST[col + 6] = 1.0
        S[4 + 2 * L + 2 * (A - 1) + o, col + 7] = 1
    return base, S, CST.reshape(1, NF), W


_BASE_NP, _S_NP, _CST_NP, _W_NP = _consts()


def _body(obs_ref, s_ref, w_ref, cst_ref, base_ref, x_ref, ei_ref, ea_ref, bv_ref):
    i = pl.program_id(0)
    obs = obs_ref[...]
    y = jnp.dot(obs, s_ref[...], preferred_element_type=jnp.float32)
    vel = obs[:, 0:2]
    recip = 1.0 / (0.001 + vel)
    ones = jnp.ones((BB, 1), jnp.float32)
    m = jnp.dot(jnp.concatenate([ones, recip], axis=1), w_ref[...],
                preferred_element_type=jnp.float32)
    x = y * m + cst_ref[...]
    x_ref[...] = x

    x0 = x[:, 0:F]
    d1 = x0 - x[:, F:2 * F]
    d2 = x0 - x[:, 2 * F:3 * F]
    e1 = jnp.sqrt(jnp.sum(d1 * d1, axis=1, keepdims=True))
    e2 = jnp.sqrt(jnp.sum(d2 * d2, axis=1, keepdims=True))
    ea_ref[...] = jnp.concatenate([e1, e2], axis=1)

    bv_ref[...] = jax.lax.broadcasted_iota(jnp.int32, (BB, N), 0) + i * BB

    ei_ref[...] = base_ref[0] + (i % 16) * (N * 32 * ROWS_PER_STEP)


@jax.jit
def _run(obs, s, w, cst, base):
    return pl.pallas_call(
        _body,
        grid=(G,),
        in_specs=[
            pl.BlockSpec((BB, OBS), lambda i: (i, 0)),
            pl.BlockSpec((OBS, NF), lambda i: (0, 0)),
            pl.BlockSpec((3, NF), lambda i: (0, 0)),
            pl.BlockSpec((1, NF), lambda i: (0, 0)),
            pl.BlockSpec((1, ROWS_PER_STEP, EC), lambda i: (i // 16, 0, 0)),
        ],
        out_specs=[
            pl.BlockSpec((BB, NF), lambda i: (i, 0)),
            pl.BlockSpec((ROWS_PER_STEP, EC), lambda i: (i, 0)),
            pl.BlockSpec((BB, 2), lambda i: (i, 0)),
            pl.BlockSpec((BB, N), lambda i: (i, 0)),
        ],
        out_shape=[
            jax.ShapeDtypeStruct((B, NF), jnp.float32),
            jax.ShapeDtypeStruct((G * ROWS_PER_STEP, EC), jnp.int32),
            jax.ShapeDtypeStruct((B, 2), jnp.float32),
            jax.ShapeDtypeStruct((B, N), jnp.int32),
        ],
    )(obs, s, w, cst, base)


def kernel(batch_observations):
    s = jnp.asarray(_S_NP)
    w = jnp.asarray(_W_NP)
    cst = jnp.asarray(_CST_NP)
    base = jnp.asarray(_BASE_NP)
    x, ei, ea, bv = _run(batch_observations, s, w, cst, base)
    return (x.reshape(B * N, F),
            ei.reshape(2, B * C),
            ea.reshape(-1),
            bv.reshape(-1))


# speedup vs baseline: 8.5685x; 1.4604x over previous
import numpy as np
import jax
import jax.numpy as jnp
from jax.experimental import pallas as pl

L = 32
A = 32
B = 4096
N = A + L
C = N * (N - 1) // 2
OBS = 4 + 2 * L + 2 * (A - 1) + (A - 1)
F = 8
NF = N * F

BB = 128
G = B // BB
GPB = BB // 32
EC = 32 * C
ROWS_PER_STEP = 8


def _consts():
    r, c = np.triu_indices(N, 1)
    rc = np.stack([r, c]).astype(np.int32)

    j = np.arange(BB * C)
    base = rc[:, j % C] + (N * (j // C)).astype(np.int32)[None, :]
    base = np.ascontiguousarray(base, dtype=np.int32)

    S = np.zeros((OBS, NF), np.float32)
    CST = np.zeros((NF,), np.float32)
    W = np.zeros((3, NF), np.float32)
    W[0, :] = 1.0
    S[2, 0] = 1; S[3, 1] = 1; S[0, 2] = 1; S[1, 3] = 1
    CST[6] = 2.0
    for l in range(L):
        col = F * (1 + l)
        sx, sy = 4 + 2 * l, 5 + 2 * l
        S[2, col] = 1; S[sx, col] = 1
        S[3, col + 1] = 1; S[sy, col + 1] = 1
        S[sx, col + 2] = 1; S[sy, col + 3] = 1
        S[sx, col + 4] = 1; S[sy, col + 5] = 1
        W[0, col + 4] = 0.0; W[1, col + 4] = 1.0
        W[0, col + 5] = 0.0; W[2, col + 5] = 1.0
    for o in range(A - 1):
        col = F * (1 + L + o)
        sx, sy = 4 + 2 * L + 2 * o, 5 + 2 * L + 2 * o
        S[2, col] = 1; S[sx, col] = 1
        S[3, col + 1] = 1; S[sy, col + 1] = 1
        S[sx, col + 2] = 1; S[sy, col + 3] = 1
        S[sx, col + 4] = 1; S[sy, col + 5] = 1
        W[0, col + 4] = 0.0; W[1, col + 4] = 1.0
        W[0, col + 5] = 0.0; W[2, col + 5] = 1.0
        CST[col + 6] = 1.0
        S[4 + 2 * L + 2 * (A - 1) + o, col + 7] = 1
    return base, S, CST.reshape(1, NF), W


_BASE_NP, _S_NP, _CST_NP, _W_NP = _consts()


def _body(obs_ref, s_ref, w_ref, cst_ref, base_ref, x_ref, ei_ref, ea_ref, bv_ref):
    i = pl.program_id(0)
    obs = obs_ref[...]
    y = jnp.dot(obs, s_ref[...], preferred_element_type=jnp.float32)
    vel = obs[:, 0:2]
    recip = 1.0 / (0.001 + vel)
    ones = jnp.ones((BB, 1), jnp.float32)
    m = jnp.dot(jnp.concatenate([ones, recip], axis=1), w_ref[...],
                preferred_element_type=jnp.float32)
    x = y * m + cst_ref[...]
    x_ref[...] = x

    x0 = x[:, 0:F]
    d1 = x0 - x[:, F:2 * F]
    d2 = x0 - x[:, 2 * F:3 * F]
    e1 = jnp.sqrt(jnp.sum(d1 * d1, axis=1, keepdims=True))
    e2 = jnp.sqrt(jnp.sum(d2 * d2, axis=1, keepdims=True))
    ea_ref[...] = jnp.concatenate([e1, e2], axis=1)

    bv_ref[...] = jax.lax.broadcasted_iota(jnp.int32, (BB, N), 0) + i * BB

    ei_ref[...] = base_ref[...] + i * (BB * N)


@jax.jit
def _run(obs, s, w, cst, base):
    return pl.pallas_call(
        _body,
        grid=(G,),
        in_specs=[
            pl.BlockSpec((BB, OBS), lambda i: (i, 0)),
            pl.BlockSpec((OBS, NF), lambda i: (0, 0)),
            pl.BlockSpec((3, NF), lambda i: (0, 0)),
            pl.BlockSpec((1, NF), lambda i: (0, 0)),
            pl.BlockSpec((2, BB * C), lambda i: (0, 0)),
        ],
        out_specs=[
            pl.BlockSpec((BB, NF), lambda i: (i, 0)),
            pl.BlockSpec((2, BB * C), lambda i: (0, i)),
            pl.BlockSpec((BB, 2), lambda i: (i, 0)),
            pl.BlockSpec((BB, N), lambda i: (i, 0)),
        ],
        out_shape=[
            jax.ShapeDtypeStruct((B, NF), jnp.float32),
            jax.ShapeDtypeStruct((2, B * C), jnp.int32),
            jax.ShapeDtypeStruct((B, 2), jnp.float32),
            jax.ShapeDtypeStruct((B, N), jnp.int32),
        ],
    )(obs, s, w, cst, base)


def kernel(batch_observations):
    s = jnp.asarray(_S_NP)
    w = jnp.asarray(_W_NP)
    cst = jnp.asarray(_CST_NP)
    base = jnp.asarray(_BASE_NP)
    x, ei, ea, bv = _run(batch_observations, s, w, cst, base)
    return (x.reshape(B * N, F), ei, ea.reshape(-1), bv.reshape(-1))


# speedup vs baseline: 22.2232x; 2.5936x over previous
import functools

import numpy as np
import jax
import jax.numpy as jnp
from jax import lax
from jax.experimental import pallas as pl
from jax.experimental.pallas import tpu as pltpu
from jax.experimental.pallas import tpu_sc as plsc

L = 32
A = 32
B = 4096
N = A + L
C = N * (N - 1) // 2
OBS = 4 + 2 * L + 2 * (A - 1) + (A - 1)
F = 8

BB = 128
G = B // BB

_NW = 32
_BPW = B // _NW
_CB = 16
_CHUNKS = _BPW // _CB
_NT = B * N * F // 1024


def _edge_base():
    r, c = np.triu_indices(N, 1)
    rc = np.stack([r, c]).astype(np.int32)
    j = np.arange(BB * C)
    base = rc[:, j % C] + (N * (j // C)).astype(np.int32)[None, :]
    return np.ascontiguousarray(base, dtype=np.int32)


def _sc_tables():
    def rel_col(k, axis):
        if k == 0:
            return 0
        if k <= L:
            return 4 + 2 * (k - 1) + axis
        return 4 + 2 * L + 2 * (k - 1 - L) + axis

    idx = np.zeros((7, 4, 16), np.int32)
    for c in range(4):
        for lane in range(16):
            k = 16 * c + lane
            idx[0, c, lane] = rel_col(k, 0)
            idx[1, c, lane] = rel_col(k, 1)
            idx[2, c, lane] = rel_col(k, 0) if k else 0
            idx[3, c, lane] = rel_col(k, 1) if k else 1
            idx[4, c, lane] = rel_col(k, 0)
            idx[5, c, lane] = rel_col(k, 1)
            idx[6, c, lane] = 4 + 2 * L + 2 * (A - 1) + (k - 1 - L) if k > L else 0
    f32 = np.zeros((3, 16), np.float32)
    f32[0] = 1.0
    f32[0, 0] = 0.0
    f32[1, 0] = 2.0
    f32[2] = 1.0
    f32[2, 0] = 0.0
    return idx.reshape(-1), f32.reshape(-1)


_BASE_NP = _edge_base()
_IDX_NP, _F32_NP = _sc_tables()


def _tc_body(obs_ref, base_ref, ei_ref, ea_ref, bv_ref):
    i = pl.program_id(0)
    obs = obs_ref[...]
    velx, vely = obs[:, 0:1], obs[:, 1:2]
    posx, posy = obs[:, 2:3], obs[:, 3:4]
    rx = 1.0 / (0.001 + velx)
    ry = 1.0 / (0.001 + vely)

    def edge(cx):
        relx, rely = obs[:, cx:cx + 1], obs[:, cx + 1:cx + 2]
        ax, ay = posx + relx, posy + rely
        d0x, d0y = posx - ax, posy - ay
        d1x, d1y = velx - relx, vely - rely
        d2x, d2y = relx * rx, rely * ry
        s = (d0x * d0x + d0y * d0y + d1x * d1x + d1y * d1y
             + d2x * d2x + d2y * d2y + 4.0)
        return jnp.sqrt(s)

    ea_ref[...] = jnp.concatenate([edge(4), edge(6)], axis=1)
    bv_ref[...] = jax.lax.broadcasted_iota(jnp.int32, (BB, N), 0) + i * BB
    ei_ref[...] = base_ref[...] + i * (BB * N)


def _run_tc(obs, base):
    return pl.pallas_call(
        _tc_body,
        grid=(G,),
        in_specs=[
            pl.BlockSpec((BB, OBS), lambda i: (i, 0)),
            pl.BlockSpec((2, BB * C), lambda i: (0, 0)),
        ],
        out_specs=[
            pl.BlockSpec((2, BB * C), lambda i: (0, i)),
            pl.BlockSpec((BB, 2), lambda i: (i, 0)),
            pl.BlockSpec((BB, N), lambda i: (i, 0)),
        ],
        out_shape=[
            jax.ShapeDtypeStruct((2, B * C), jnp.int32),
            jax.ShapeDtypeStruct((B, 2), jnp.float32),
            jax.ShapeDtypeStruct((B, N), jnp.int32),
        ],
    )(obs, base)


def _run_sc(obs, idx_tab, f32_tab):
    mesh = plsc.VectorSubcoreMesh(core_axis_name="c", subcore_axis_name="s")

    @functools.partial(
        pl.kernel,
        out_type=jax.ShapeDtypeStruct((B * N * F,), jnp.float32),
        mesh=mesh,
        compiler_params=pltpu.CompilerParams(
            needs_layout_passes=False, use_tc_tiling_on_sc=False),
        scratch_types=[
            pltpu.VMEM((_CB * OBS,), jnp.float32),
            pltpu.VMEM((7 * 4 * 16,), jnp.int32),
            pltpu.VMEM((3 * 16,), jnp.float32),
            pltpu.VMEM((8 * 8 * 128,), jnp.float32),
        ],
    )
    def k(obs_hbm, idx_hbm, f32_hbm, out_hbm, obs_v, idx_v, f32_v, buf_v):
        wid = lax.axis_index("s") * 2 + lax.axis_index("c")
        pltpu.sync_copy(idx_hbm, idx_v)
        pltpu.sync_copy(f32_hbm, f32_v)
        sel0 = f32_v[pl.ds(0, 16)]
        f6c0 = f32_v[pl.ds(16, 16)]
        f6c2 = f32_v[pl.ds(32, 16)]
        zeros16 = jnp.zeros((16,), jnp.float32)
        ones16 = jnp.full((16,), 1.0, jnp.float32)

        def chunk_body(ch, carry):
            b0 = (wid * _BPW + ch * _CB) * OBS
            pltpu.sync_copy(obs_hbm.at[pl.ds(b0, _CB * OBS)], obs_v)

            def batch_body(bl, carry2):
                roff = bl * OBS
                splat = jnp.full((16,), 0, jnp.int32) + roff
                velx = plsc.load_gather(obs_v, [splat])
                vely = plsc.load_gather(obs_v, [splat + 1])
                posx = plsc.load_gather(obs_v, [splat + 2])
                posy = plsc.load_gather(obs_v, [splat + 3])
                rx = 1.0 / (0.001 + velx)
                ry = 1.0 / (0.001 + vely)
                bbase = (bl // 2) * 1024 + (bl % 2) * 64
                for c in range(4):
                    off = bbase + 16 * c

                    def g(slot, c=c, roff=roff):
                        cid = idx_v[pl.ds(16 * (slot * 4 + c), 16)]
                        return plsc.load_gather(obs_v, [cid + roff])

                    a0, a1 = g(0), g(1)
                    if c == 0:
                        a0, a1 = a0 * sel0, a1 * sel0
                    buf_v[pl.ds(off, 16)] = a0 + posx
                    buf_v[pl.ds(off + 128, 16)] = a1 + posy
                    buf_v[pl.ds(off + 256, 16)] = g(2)
                    buf_v[pl.ds(off + 384, 16)] = g(3)
                    a4, a5 = g(4), g(5)
                    if c == 0:
                        a4, a5 = a4 * sel0, a5 * sel0
                    buf_v[pl.ds(off + 512, 16)] = a4 * rx
                    buf_v[pl.ds(off + 640, 16)] = a5 * ry
                    if c == 0:
                        buf_v[pl.ds(off + 768, 16)] = f6c0
                    elif c == 1:
                        buf_v[pl.ds(off + 768, 16)] = zeros16
                    elif c == 2:
                        buf_v[pl.ds(off + 768, 16)] = f6c2
                    else:
                        buf_v[pl.ds(off + 768, 16)] = ones16
                    if c < 2:
                        buf_v[pl.ds(off + 896, 16)] = zeros16
                    else:
                        a7 = g(6)
                        if c == 2:
                            a7 = a7 * sel0
                        buf_v[pl.ds(off + 896, 16)] = a7
                return carry2

            lax.fori_loop(0, _CB, batch_body, 0)
            pltpu.sync_copy(buf_v, out_hbm.at[pl.ds((wid * 64 + ch * 8) * 1024, 8 * 1024)])
            return carry

        lax.fori_loop(0, _CHUNKS, chunk_body, 0)

    return k(obs, idx_tab, f32_tab)


def kernel(batch_observations):
    base = jnp.asarray(_BASE_NP)
    idx_tab = jnp.asarray(_IDX_NP)
    f32_tab = jnp.asarray(_F32_NP)
    ei, ea, bv = _run_tc(batch_observations, base)
    xt = _run_sc(batch_observations.reshape(-1), idx_tab, f32_tab)
    x = jnp.transpose(xt.reshape(_NT, 8, 128), (0, 2, 1)).reshape(B * N, F)
    return x, ei, ea.reshape(-1), bv.reshape(-1)


# speedup vs baseline: 22.9082x; 1.0308x over previous
import functools

import numpy as np
import jax
import jax.numpy as jnp
from jax import lax
from jax.experimental import pallas as pl
from jax.experimental.pallas import tpu as pltpu
from jax.experimental.pallas import tpu_sc as plsc

L = 32
A = 32
B = 4096
N = A + L
C = N * (N - 1) // 2
OBS = 4 + 2 * L + 2 * (A - 1) + (A - 1)
F = 8

BB = 256
G = B // BB

_NW = 32
_BPW = B // _NW
_CB = 16
_CHUNKS = _BPW // _CB
_NT = B * N * F // 1024


def _edge_base():
    r, c = np.triu_indices(N, 1)
    rc = np.stack([r, c]).astype(np.int32)
    j = np.arange(BB * C)
    base = rc[:, j % C] + (N * (j // C)).astype(np.int32)[None, :]
    return np.ascontiguousarray(base, dtype=np.int32)


def _sc_tables():
    def rel_col(k, axis):
        if k == 0:
            return 0
        if k <= L:
            return 4 + 2 * (k - 1) + axis
        return 4 + 2 * L + 2 * (k - 1 - L) + axis

    idx = np.zeros((7, 4, 16), np.int32)
    for c in range(4):
        for lane in range(16):
            k = 16 * c + lane
            idx[0, c, lane] = rel_col(k, 0)
            idx[1, c, lane] = rel_col(k, 1)
            idx[2, c, lane] = rel_col(k, 0) if k else 0
            idx[3, c, lane] = rel_col(k, 1) if k else 1
            idx[4, c, lane] = rel_col(k, 0)
            idx[5, c, lane] = rel_col(k, 1)
            idx[6, c, lane] = 4 + 2 * L + 2 * (A - 1) + (k - 1 - L) if k > L else 0
    f32 = np.zeros((3, 16), np.float32)
    f32[0] = 1.0
    f32[0, 0] = 0.0
    f32[1, 0] = 2.0
    f32[2] = 1.0
    f32[2, 0] = 0.0
    return idx.reshape(-1), f32.reshape(-1)


_BASE_NP = _edge_base()
_IDX_NP, _F32_NP = _sc_tables()


def _tc_body(obs_ref, base_ref, ei_ref, ea_ref, bv_ref):
    i = pl.program_id(0)
    obs = obs_ref[...]
    velx, vely = obs[:, 0:1], obs[:, 1:2]
    posx, posy = obs[:, 2:3], obs[:, 3:4]
    rx = 1.0 / (0.001 + velx)
    ry = 1.0 / (0.001 + vely)

    def edge(cx):
        relx, rely = obs[:, cx:cx + 1], obs[:, cx + 1:cx + 2]
        ax, ay = posx + relx, posy + rely
        d0x, d0y = posx - ax, posy - ay
        d1x, d1y = velx - relx, vely - rely
        d2x, d2y = relx * rx, rely * ry
        s = (d0x * d0x + d0y * d0y + d1x * d1x + d1y * d1y
             + d2x * d2x + d2y * d2y + 4.0)
        return jnp.sqrt(s)

    ea_ref[...] = jnp.concatenate([edge(4), edge(6)], axis=1)
    bv_ref[...] = jax.lax.broadcasted_iota(jnp.int32, (BB, N), 0) + i * BB
    ei_ref[...] = base_ref[...] + i * (BB * N)


def _run_tc(obs, base):
    return pl.pallas_call(
        _tc_body,
        grid=(G,),
        in_specs=[
            pl.BlockSpec((BB, 8), lambda i: (i, 0)),
            pl.BlockSpec((2, BB * C), lambda i: (0, 0)),
        ],
        out_specs=[
            pl.BlockSpec((2, BB * C), lambda i: (0, i)),
            pl.BlockSpec((BB, 2), lambda i: (i, 0)),
            pl.BlockSpec((BB, N), lambda i: (i, 0)),
        ],
        out_shape=[
            jax.ShapeDtypeStruct((2, B * C), jnp.int32),
            jax.ShapeDtypeStruct((B, 2), jnp.float32),
            jax.ShapeDtypeStruct((B, N), jnp.int32),
        ],
    )(obs, base)


def _run_sc(obs, idx_tab, f32_tab):
    mesh = plsc.VectorSubcoreMesh(core_axis_name="c", subcore_axis_name="s")

    @functools.partial(
        pl.kernel,
        out_type=jax.ShapeDtypeStruct((B * N * F,), jnp.float32),
        mesh=mesh,
        compiler_params=pltpu.CompilerParams(
            needs_layout_passes=False, use_tc_tiling_on_sc=False),
        scratch_types=[
            pltpu.VMEM((_CB * OBS,), jnp.float32),
            pltpu.VMEM((7 * 4 * 16,), jnp.int32),
            pltpu.VMEM((3 * 16,), jnp.float32),
            pltpu.VMEM((8 * 8 * 128,), jnp.float32),
        ],
    )
    def k(obs_hbm, idx_hbm, f32_hbm, out_hbm, obs_v, idx_v, f32_v, buf_v):
        wid = lax.axis_index("s") * 2 + lax.axis_index("c")
        pltpu.sync_copy(idx_hbm, idx_v)
        pltpu.sync_copy(f32_hbm, f32_v)
        sel0 = f32_v[pl.ds(0, 16)]
        f6c0 = f32_v[pl.ds(16, 16)]
        f6c2 = f32_v[pl.ds(32, 16)]
        zeros16 = jnp.zeros((16,), jnp.float32)
        ones16 = jnp.full((16,), 1.0, jnp.float32)

        def chunk_body(ch, carry):
            b0 = (wid * _BPW + ch * _CB) * OBS
            pltpu.sync_copy(obs_hbm.at[pl.ds(b0, _CB * OBS)], obs_v)

            def batch_body(bl, carry2):
                roff = bl * OBS
                splat = jnp.full((16,), 0, jnp.int32) + roff
                velx = plsc.load_gather(obs_v, [splat])
                vely = plsc.load_gather(obs_v, [splat + 1])
                posx = plsc.load_gather(obs_v, [splat + 2])
                posy = plsc.load_gather(obs_v, [splat + 3])
                rx = 1.0 / (0.001 + velx)
                ry = 1.0 / (0.001 + vely)
                bbase = (bl // 2) * 1024 + (bl % 2) * 64
                for c in range(4):
                    off = bbase + 16 * c

                    def g(slot, c=c, roff=roff):
                        cid = idx_v[pl.ds(16 * (slot * 4 + c), 16)]
                        return plsc.load_gather(obs_v, [cid + roff])

                    a0, a1 = g(0), g(1)
                    if c == 0:
                        a0, a1 = a0 * sel0, a1 * sel0
                    buf_v[pl.ds(off, 16)] = a0 + posx
                    buf_v[pl.ds(off + 128, 16)] = a1 + posy
                    buf_v[pl.ds(off + 256, 16)] = g(2)
                    buf_v[pl.ds(off + 384, 16)] = g(3)
                    a4, a5 = g(4), g(5)
                    if c == 0:
                        a4, a5 = a4 * sel0, a5 * sel0
                    buf_v[pl.ds(off + 512, 16)] = a4 * rx
                    buf_v[pl.ds(off + 640, 16)] = a5 * ry
                    if c == 0:
                        buf_v[pl.ds(off + 768, 16)] = f6c0
                    elif c == 1:
                        buf_v[pl.ds(off + 768, 16)] = zeros16
                    elif c == 2:
                        buf_v[pl.ds(off + 768, 16)] = f6c2
                    else:
                        buf_v[pl.ds(off + 768, 16)] = ones16
                    if c < 2:
                        buf_v[pl.ds(off + 896, 16)] = zeros16
                    else:
                        a7 = g(6)
                        if c == 2:
                            a7 = a7 * sel0
                        buf_v[pl.ds(off + 896, 16)] = a7
                return carry2

            lax.fori_loop(0, _CB, batch_body, 0)
            pltpu.sync_copy(buf_v, out_hbm.at[pl.ds((wid * 64 + ch * 8) * 1024, 8 * 1024)])
            return carry

        lax.fori_loop(0, _CHUNKS, chunk_body, 0)

    return k(obs, idx_tab, f32_tab)


def kernel(batch_observations):
    base = jnp.asarray(_BASE_NP)
    idx_tab = jnp.asarray(_IDX_NP)
    f32_tab = jnp.asarray(_F32_NP)
    ei, ea, bv = _run_tc(batch_observations[:, :8], base)
    xt = _run_sc(batch_observations.reshape(-1), idx_tab, f32_tab)
    x = jnp.transpose(xt.reshape(_NT, 8, 128), (0, 2, 1)).reshape(B * N, F)
    return x, ei, ea.reshape(-1), bv.reshape(-1)


# speedup vs baseline: 24.5029x; 1.0696x over previous
import functools

import numpy as np
import jax
import jax.numpy as jnp
from jax import lax
from jax.experimental import pallas as pl
from jax.experimental.pallas import tpu as pltpu
from jax.experimental.pallas import tpu_sc as plsc

L = 32
A = 32
B = 4096
N = A + L
C = N * (N - 1) // 2
OBS = 4 + 2 * L + 2 * (A - 1) + (A - 1)
F = 8

BB = 256
G = B // BB

_NW = 32
_BPW = B // _NW
_CB = 16
_CHUNKS = _BPW // _CB
_NT = B * N * F // 1024


def _edge_base():
    r, c = np.triu_indices(N, 1)
    rc = np.stack([r, c]).astype(np.int32)
    j = np.arange(BB * C)
    base = rc[:, j % C] + (N * (j // C)).astype(np.int32)[None, :]
    return np.ascontiguousarray(base, dtype=np.int32)


def _sc_tables():
    def rel_col(k, axis):
        if k == 0:
            return 0
        if k <= L:
            return 4 + 2 * (k - 1) + axis
        return 4 + 2 * L + 2 * (k - 1 - L) + axis

    idx = np.zeros((7, 4, 16), np.int32)
    for c in range(4):
        for lane in range(16):
            k = 16 * c + lane
            idx[0, c, lane] = rel_col(k, 0)
            idx[1, c, lane] = rel_col(k, 1)
            idx[2, c, lane] = rel_col(k, 0) if k else 0
            idx[3, c, lane] = rel_col(k, 1) if k else 1
            idx[4, c, lane] = rel_col(k, 0)
            idx[5, c, lane] = rel_col(k, 1)
            idx[6, c, lane] = 4 + 2 * L + 2 * (A - 1) + (k - 1 - L) if k > L else 0
    f32 = np.zeros((3, 16), np.float32)
    f32[0] = 1.0
    f32[0, 0] = 0.0
    f32[1, 0] = 2.0
    f32[2] = 1.0
    f32[2, 0] = 0.0
    return idx.reshape(-1), f32.reshape(-1)


_BASE_NP = _edge_base()
_IDX_NP, _F32_NP = _sc_tables()


def _tc_body(obs_ref, base_ref, ei_ref, ea_ref, bv_ref):
    i = pl.program_id(0)
    obs = obs_ref[...]
    velx, vely = obs[:, 0:1], obs[:, 1:2]
    posx, posy = obs[:, 2:3], obs[:, 3:4]
    rx = 1.0 / (0.001 + velx)
    ry = 1.0 / (0.001 + vely)

    def edge(cx):
        relx, rely = obs[:, cx:cx + 1], obs[:, cx + 1:cx + 2]
        ax, ay = posx + relx, posy + rely
        d0x, d0y = posx - ax, posy - ay
        d1x, d1y = velx - relx, vely - rely
        d2x, d2y = relx * rx, rely * ry
        s = (d0x * d0x + d0y * d0y + d1x * d1x + d1y * d1y
             + d2x * d2x + d2y * d2y + 4.0)
        return jnp.sqrt(s)

    ea_ref[...] = jnp.concatenate([edge(4), edge(6)], axis=1)
    bv_ref[...] = jax.lax.broadcasted_iota(jnp.int32, (BB, N), 0) + i * BB
    ei_ref[...] = base_ref[...] + i * (BB * N)


def _run_tc(obs, base):
    return pl.pallas_call(
        _tc_body,
        grid=(G,),
        in_specs=[
            pl.BlockSpec((BB, 8), lambda i: (i, 0)),
            pl.BlockSpec((2, BB * C), lambda i: (0, 0)),
        ],
        out_specs=[
            pl.BlockSpec((2, BB * C), lambda i: (0, i)),
            pl.BlockSpec((BB, 2), lambda i: (i, 0)),
            pl.BlockSpec((BB, N), lambda i: (i, 0)),
        ],
        out_shape=[
            jax.ShapeDtypeStruct((2, B * C), jnp.int32),
            jax.ShapeDtypeStruct((B, 2), jnp.float32),
            jax.ShapeDtypeStruct((B, N), jnp.int32),
        ],
    )(obs, base)


def _run_sc(obs, idx_tab, f32_tab):
    mesh = plsc.VectorSubcoreMesh(core_axis_name="c", subcore_axis_name="s")

    @functools.partial(
        pl.kernel,
        out_type=jax.ShapeDtypeStruct((B * N * F,), jnp.float32),
        mesh=mesh,
        compiler_params=pltpu.CompilerParams(
            needs_layout_passes=False, use_tc_tiling_on_sc=False),
        scratch_types=[
            pltpu.VMEM((_CB * OBS,), jnp.float32),
            pltpu.VMEM((_CB * OBS,), jnp.float32),
            pltpu.VMEM((7 * 4 * 16,), jnp.int32),
            pltpu.VMEM((3 * 16,), jnp.float32),
            pltpu.VMEM((8 * 8 * 128,), jnp.float32),
            pltpu.VMEM((8 * 8 * 128,), jnp.float32),
            pltpu.SemaphoreType.DMA,
            pltpu.SemaphoreType.DMA,
            pltpu.SemaphoreType.DMA,
            pltpu.SemaphoreType.DMA,
        ],
    )
    def k(obs_hbm, idx_hbm, f32_hbm, out_hbm,
          obs_v0, obs_v1, idx_v, f32_v, buf_v0, buf_v1, os0, os1, ws0, ws1):
        wid = lax.axis_index("s") * 2 + lax.axis_index("c")
        pltpu.sync_copy(idx_hbm, idx_v)
        pltpu.sync_copy(f32_hbm, f32_v)
        sel0 = f32_v[pl.ds(0, 16)]
        f6c0 = f32_v[pl.ds(16, 16)]
        f6c2 = f32_v[pl.ds(32, 16)]
        zeros16 = jnp.zeros((16,), jnp.float32)
        ones16 = jnp.full((16,), 1.0, jnp.float32)
        idx_tabs = [[idx_v[pl.ds(16 * (slot * 4 + c), 16)] for c in range(4)]
                    for slot in range(7)]
        obs_bufs = [obs_v0, obs_v1]
        out_bufs = [buf_v0, buf_v1]
        osems = [os0, os1]
        wsems = [ws0, ws1]

        def obs_fetch(ch, ob, sem):
            b0 = (wid * _BPW + ch * _CB) * OBS
            return pltpu.async_copy(obs_hbm.at[pl.ds(b0, _CB * OBS)], ob, sem)

        def compute_chunk(ob, buf_v):
            def batch_body(bl, carry2):
                roff = bl * OBS
                splat = jnp.full((16,), 0, jnp.int32) + roff
                velx = plsc.load_gather(ob, [splat])
                vely = plsc.load_gather(ob, [splat + 1])
                posx = plsc.load_gather(ob, [splat + 2])
                posy = plsc.load_gather(ob, [splat + 3])
                rx = 1.0 / (0.001 + velx)
                ry = 1.0 / (0.001 + vely)
                bbase = (bl // 2) * 1024 + (bl % 2) * 64
                for c in range(4):
                    off = bbase + 16 * c

                    def g(slot, c=c, roff=roff):
                        return plsc.load_gather(ob, [idx_tabs[slot][c] + roff])

                    a0, a1 = g(0), g(1)
                    if c == 0:
                        a0, a1 = a0 * sel0, a1 * sel0
                    buf_v[pl.ds(off, 16)] = a0 + posx
                    buf_v[pl.ds(off + 128, 16)] = a1 + posy
                    buf_v[pl.ds(off + 256, 16)] = g(2)
                    buf_v[pl.ds(off + 384, 16)] = g(3)
                    a4, a5 = g(4), g(5)
                    if c == 0:
                        a4, a5 = a4 * sel0, a5 * sel0
                    buf_v[pl.ds(off + 512, 16)] = a4 * rx
                    buf_v[pl.ds(off + 640, 16)] = a5 * ry
                    if c == 0:
                        buf_v[pl.ds(off + 768, 16)] = f6c0
                    elif c == 1:
                        buf_v[pl.ds(off + 768, 16)] = zeros16
                    elif c == 2:
                        buf_v[pl.ds(off + 768, 16)] = f6c2
                    else:
                        buf_v[pl.ds(off + 768, 16)] = ones16
                    if c < 2:
                        buf_v[pl.ds(off + 896, 16)] = zeros16
                    else:
                        a7 = g(6)
                        if c == 2:
                            a7 = a7 * sel0
                        buf_v[pl.ds(off + 896, 16)] = a7
                return carry2

            lax.fori_loop(0, _CB, batch_body, 0)

        oh = [obs_fetch(0, obs_bufs[0], osems[0]), None]
        wh = [None, None]
        for ch in range(_CHUNKS):
            s = ch % 2
            if ch + 1 < _CHUNKS:
                oh[1 - s] = obs_fetch(ch + 1, obs_bufs[1 - s], osems[1 - s])
            oh[s].wait()
            if wh[s] is not None:
                wh[s].wait()
            compute_chunk(obs_bufs[s], out_bufs[s])
            wh[s] = pltpu.async_copy(
                out_bufs[s],
                out_hbm.at[pl.ds((wid * 64 + ch * 8) * 1024, 8 * 1024)],
                wsems[s])
        wh[0].wait()
        wh[1].wait()

    return k(obs, idx_tab, f32_tab)


def kernel(batch_observations):
    base = jnp.asarray(_BASE_NP)
    idx_tab = jnp.asarray(_IDX_NP)
    f32_tab = jnp.asarray(_F32_NP)
    ei, ea, bv = _run_tc(batch_observations[:, :8], base)
    xt = _run_sc(batch_observations.reshape(-1), idx_tab, f32_tab)
    x = jnp.transpose(xt.reshape(_NT, 8, 128), (0, 2, 1)).reshape(B * N, F)
    return x, ei, ea.reshape(-1), bv.reshape(-1)
